# baseline (speedup 1.0000x reference)
import functools

import jax
import jax.numpy as jnp
from jax import lax
from jax.experimental import pallas as pl
from jax.experimental.pallas import tpu as pltpu
from jax.experimental.pallas import tpu_sc as plsc

TOPK = 2
EPS = 1e-20
E = 8
LANES = 16


def _scores_kernel(x_ref, w_ref, st_ref):
    x = x_ref[...]
    w = w_ref[...]
    logits = jax.lax.dot_general(
        x, w, (((1,), (1,)), ((), ())),
        preferred_element_type=jnp.float32)
    m = jnp.max(logits, axis=-1, keepdims=True)
    e = jnp.exp(logits - m)
    s = e / jnp.sum(e, axis=-1, keepdims=True)
    st_ref[...] = s.T


@functools.partial(jax.jit, static_argnames=("tile", "n_chunks", "chunk"))
def _scores_t(x2d, w, tile, n_chunks=1, chunk=0):
    S, H = x2d.shape
    cs = S // n_chunks
    off = chunk * (cs // tile)
    return pl.pallas_call(
        _scores_kernel,
        grid=(cs // tile,),
        in_specs=[
            pl.BlockSpec((tile, H), lambda i: (i + off, 0)),
            pl.BlockSpec((E, H), lambda i: (0, 0)),
        ],
        out_specs=pl.BlockSpec((E, tile), lambda i: (0, i)),
        out_shape=jax.ShapeDtypeStruct((E, cs), jnp.float32),
        compiler_params=pltpu.CompilerParams(
            dimension_semantics=("parallel",)),
    )(x2d, w)


def _dg(v, idx):
    return v.at[idx].get(mode="promise_in_bounds")


def _make_sc_router(S):
    info = plsc.get_sparse_core_info()
    NC, NS = info.num_cores, info.num_subcores
    NW = NC * NS
    rows = S // NW
    chunks = rows // LANES

    mesh = plsc.VectorSubcoreMesh(core_axis_name="c", subcore_axis_name="s")

    @functools.partial(
        pl.kernel,
        mesh=mesh,
        out_type=[
            jax.ShapeDtypeStruct((S,), jnp.int32),
            jax.ShapeDtypeStruct((S,), jnp.int32),
            jax.ShapeDtypeStruct((S,), jnp.float32),
            jax.ShapeDtypeStruct((S,), jnp.float32),
        ],
        scratch_types=[
            pltpu.VMEM((E, rows), jnp.float32),
            pltpu.VMEM((rows,), jnp.int32),
            pltpu.VMEM((rows,), jnp.int32),
            pltpu.VMEM((rows,), jnp.float32),
            pltpu.VMEM((rows,), jnp.float32),
        ],
    )
    def _sc_router(st_hbm, i1_hbm, i2_hbm, w1_hbm, w2_hbm,
                   s_v, i1_v, i2_v, w1_v, w2_v):
        wid = lax.axis_index("s") * NC + lax.axis_index("c")
        base = wid * rows
        pltpu.sync_copy(st_hbm.at[:, pl.ds(base, rows)], s_v)
        zeros = jnp.zeros((LANES,), jnp.int32)
        neg = jnp.full((LANES,), -jnp.inf, jnp.float32)
        evs = [jnp.full((LANES,), e, jnp.int32) for e in range(E)]
        for j in range(chunks):
            vs = [s_v[e, pl.ds(j * LANES, LANES)] for e in range(E)]
            m1 = vs[0]
            i1 = zeros
            for e in range(1, E):
                gt = vs[e] > m1
                m1 = jnp.where(gt, vs[e], m1)
                i1 = jnp.where(gt, evs[e], i1)
            m2 = neg
            i2 = zeros
            for e in range(E):
                cand = jnp.where(i1 == evs[e], neg, vs[e])
                gt = cand > m2
                m2 = jnp.where(gt, cand, m2)
                i2 = jnp.where(gt, evs[e], i2)
            inv = 1.0 / (m1 + m2 + EPS)
            sl = pl.ds(j * LANES, LANES)
            i1_v[sl] = i1
            i2_v[sl] = i2
            w1_v[sl] = m1 * inv
            w2_v[sl] = m2 * inv
        row_sl = pl.ds(base, rows)
        pltpu.sync_copy(i1_v, i1_hbm.at[row_sl])
        pltpu.sync_copy(i2_v, i2_hbm.at[row_sl])
        pltpu.sync_copy(w1_v, w1_hbm.at[row_sl])
        pltpu.sync_copy(w2_v, w2_hbm.at[row_sl])

    return _sc_router


N_CHUNKS = 2


@jax.jit
def _route(x2d, w):
    S = x2d.shape[0]
    cs = S // N_CHUNKS
    router = _make_sc_router(cs)
    parts = []
    for c in range(N_CHUNKS):
        st = _scores_t(x2d, w, tile=1024 if cs >= 1024 else cs,
                       n_chunks=N_CHUNKS, chunk=c)
        parts.append(router(st))
    i1, i2, w1, w2 = (jnp.concatenate([p[k] for p in parts]) for k in range(4))
    return (jnp.stack([i1, i2], axis=-1), jnp.stack([w1, w2], axis=-1))


def kernel(hidden_states, weight):
    bsz, seq_len, h = hidden_states.shape
    x2d = hidden_states.reshape(-1, h).astype(jnp.float32)
    topk_idx, topk_weight = _route(x2d, weight.astype(jnp.float32))
    return (topk_idx, topk_weight)


# speedup vs baseline: 1.0734x; 1.0734x over previous
import functools

import jax
import jax.numpy as jnp
from jax import lax
from jax.experimental import pallas as pl
from jax.experimental.pallas import tpu as pltpu
from jax.experimental.pallas import tpu_sc as plsc

TOPK = 2
EPS = 1e-20
E = 8
LANES = 16


def _scores_kernel(xa_ref, xb_ref, w_ref, st_ref):
    w = w_ref[...]
    hh = w.shape[1] // 2
    la = jax.lax.dot_general(
        xa_ref[...], w[:, :hh], (((1,), (1,)), ((), ())),
        preferred_element_type=jnp.float32)
    lb = jax.lax.dot_general(
        xb_ref[...], w[:, hh:], (((1,), (1,)), ((), ())),
        preferred_element_type=jnp.float32)
    logits = la + lb
    m = jnp.max(logits, axis=-1, keepdims=True)
    e = jnp.exp(logits - m)
    s = e / jnp.sum(e, axis=-1, keepdims=True)
    st_ref[...] = s.T


@functools.partial(jax.jit, static_argnames=("tile",))
def _scores_t(x2d, w, tile):
    S, H = x2d.shape
    return pl.pallas_call(
        _scores_kernel,
        grid=(S // tile,),
        in_specs=[
            pl.BlockSpec((tile, H // 2), lambda i: (i, 0)),
            pl.BlockSpec((tile, H // 2), lambda i: (i, 1)),
            pl.BlockSpec((E, H), lambda i: (0, 0)),
        ],
        out_specs=pl.BlockSpec((E, tile), lambda i: (0, i)),
        out_shape=jax.ShapeDtypeStruct((E, S), jnp.float32),
        compiler_params=pltpu.CompilerParams(
            dimension_semantics=("parallel",)),
    )(x2d, x2d, w)


def _make_sc_router(S):
    info = plsc.get_sparse_core_info()
    NC, NS = info.num_cores, info.num_subcores
    NW = NC * NS
    rows = S // NW
    chunks = rows // LANES

    mesh = plsc.VectorSubcoreMesh(core_axis_name="c", subcore_axis_name="s")

    @functools.partial(
        pl.kernel,
        mesh=mesh,
        out_type=[
            jax.ShapeDtypeStruct((S,), jnp.int32),
            jax.ShapeDtypeStruct((S,), jnp.int32),
            jax.ShapeDtypeStruct((S,), jnp.float32),
            jax.ShapeDtypeStruct((S,), jnp.float32),
        ],
        scratch_types=[
            pltpu.VMEM((E, rows), jnp.float32),
            pltpu.VMEM((rows,), jnp.int32),
            pltpu.VMEM((rows,), jnp.int32),
            pltpu.VMEM((rows,), jnp.float32),
            pltpu.VMEM((rows,), jnp.float32),
        ],
    )
    def _sc_router(st_hbm, i1_hbm, i2_hbm, w1_hbm, w2_hbm,
                   s_v, i1_v, i2_v, w1_v, w2_v):
        wid = lax.axis_index("s") * NC + lax.axis_index("c")
        base = wid * rows
        pltpu.sync_copy(st_hbm.at[:, pl.ds(base, rows)], s_v)
        zeros = jnp.zeros((LANES,), jnp.int32)
        neg = jnp.full((LANES,), -jnp.inf, jnp.float32)
        evs = [jnp.full((LANES,), e, jnp.int32) for e in range(E)]
        for j in range(chunks):
            vs = [s_v[e, pl.ds(j * LANES, LANES)] for e in range(E)]
            m1 = vs[0]
            i1 = zeros
            for e in range(1, E):
                gt = vs[e] > m1
                m1 = jnp.where(gt, vs[e], m1)
                i1 = jnp.where(gt, evs[e], i1)
            m2 = neg
            i2 = zeros
            for e in range(E):
                cand = jnp.where(i1 == evs[e], neg, vs[e])
                gt = cand > m2
                m2 = jnp.where(gt, cand, m2)
                i2 = jnp.where(gt, evs[e], i2)
            inv = 1.0 / (m1 + m2 + EPS)
            sl = pl.ds(j * LANES, LANES)
            i1_v[sl] = i1
            i2_v[sl] = i2
            w1_v[sl] = m1 * inv
            w2_v[sl] = m2 * inv
        row_sl = pl.ds(base, rows)
        pltpu.sync_copy(i1_v, i1_hbm.at[row_sl])
        pltpu.sync_copy(i2_v, i2_hbm.at[row_sl])
        pltpu.sync_copy(w1_v, w1_hbm.at[row_sl])
        pltpu.sync_copy(w2_v, w2_hbm.at[row_sl])

    return _sc_router


@jax.jit
def _route(x2d, w):
    S = x2d.shape[0]
    st = _scores_t(x2d, w, tile=1024)
    i1, i2, w1, w2 = _make_sc_router(S)(st)
    return (jnp.stack([i1, i2], axis=-1), jnp.stack([w1, w2], axis=-1))


def kernel(hidden_states, weight):
    bsz, seq_len, h = hidden_states.shape
    x2d = hidden_states.reshape(-1, h).astype(jnp.float32)
    topk_idx, topk_weight = _route(x2d, weight.astype(jnp.float32))
    return (topk_idx, topk_weight)


# speedup vs baseline: 1.0761x; 1.0025x over previous
import functools

import jax
import jax.numpy as jnp
from jax import lax
from jax.experimental import pallas as pl
from jax.experimental.pallas import tpu as pltpu
from jax.experimental.pallas import tpu_sc as plsc

TOPK = 2
EPS = 1e-20
E = 8
LANES = 16


def _scores_kernel(x_ref, w_ref, st_ref):
    x = x_ref[...]
    w = w_ref[...]
    logits = jax.lax.dot_general(
        x, w, (((1,), (1,)), ((), ())),
        preferred_element_type=jnp.float32)
    m = jnp.max(logits, axis=-1, keepdims=True)
    e = jnp.exp(logits - m)
    s = e / jnp.sum(e, axis=-1, keepdims=True)
    tile = s.shape[0]
    wpt = tile // 256
    st_ref[...] = s.reshape(wpt, 256, E2).transpose(0, 2, 1)


E2 = 8


@functools.partial(jax.jit, static_argnames=("tile",))
def _scores_t(x2d, w, tile):
    S, H = x2d.shape
    wpt = tile // 256
    return pl.pallas_call(
        _scores_kernel,
        grid=(S // tile,),
        in_specs=[
            pl.BlockSpec((tile, H), lambda i: (i, 0)),
            pl.BlockSpec((E, H), lambda i: (0, 0)),
        ],
        out_specs=pl.BlockSpec((wpt, E, 256), lambda i: (i, 0, 0)),
        out_shape=jax.ShapeDtypeStruct((S // 256, E, 256), jnp.float32),
        compiler_params=pltpu.CompilerParams(
            dimension_semantics=("parallel",)),
    )(x2d, w)


def _make_sc_router(S):
    info = plsc.get_sparse_core_info()
    NC, NS = info.num_cores, info.num_subcores
    NW = NC * NS
    rows = S // NW
    chunks = rows // LANES

    mesh = plsc.VectorSubcoreMesh(core_axis_name="c", subcore_axis_name="s")

    @functools.partial(
        pl.kernel,
        mesh=mesh,
        out_type=[
            jax.ShapeDtypeStruct((S,), jnp.int32),
            jax.ShapeDtypeStruct((S,), jnp.int32),
            jax.ShapeDtypeStruct((S,), jnp.float32),
            jax.ShapeDtypeStruct((S,), jnp.float32),
        ],
        scratch_types=[
            pltpu.VMEM((E, rows), jnp.float32),
            pltpu.VMEM((rows,), jnp.int32),
            pltpu.VMEM((rows,), jnp.int32),
            pltpu.VMEM((rows,), jnp.float32),
            pltpu.VMEM((rows,), jnp.float32),
        ],
    )
    def _sc_router(st_hbm, i1_hbm, i2_hbm, w1_hbm, w2_hbm,
                   s_v, i1_v, i2_v, w1_v, w2_v):
        wid = lax.axis_index("s") * NC + lax.axis_index("c")
        base = wid * rows
        pltpu.sync_copy(st_hbm.at[wid], s_v)
        zeros = jnp.zeros((LANES,), jnp.int32)
        neg = jnp.full((LANES,), -jnp.inf, jnp.float32)
        evs = [jnp.full((LANES,), e, jnp.int32) for e in range(E)]
        for j in range(chunks):
            vs = [s_v[e, pl.ds(j * LANES, LANES)] for e in range(E)]
            m1 = vs[0]
            i1 = zeros
            for e in range(1, E):
                gt = vs[e] > m1
                m1 = jnp.where(gt, vs[e], m1)
                i1 = jnp.where(gt, evs[e], i1)
            m2 = neg
            i2 = zeros
            for e in range(E):
                cand = jnp.where(i1 == evs[e], neg, vs[e])
                gt = cand > m2
                m2 = jnp.where(gt, cand, m2)
                i2 = jnp.where(gt, evs[e], i2)
            inv = 1.0 / (m1 + m2 + EPS)
            sl = pl.ds(j * LANES, LANES)
            i1_v[sl] = i1
            i2_v[sl] = i2
            w1_v[sl] = m1 * inv
            w2_v[sl] = m2 * inv
        row_sl = pl.ds(base, rows)
        pltpu.sync_copy(i1_v, i1_hbm.at[row_sl])
        pltpu.sync_copy(i2_v, i2_hbm.at[row_sl])
        pltpu.sync_copy(w1_v, w1_hbm.at[row_sl])
        pltpu.sync_copy(w2_v, w2_hbm.at[row_sl])

    return _sc_router


@jax.jit
def _route(x2d, w):
    S = x2d.shape[0]
    st = _scores_t(x2d, w, tile=1024)
    i1, i2, w1, w2 = _make_sc_router(S)(st)
    return (jnp.stack([i1, i2], axis=-1), jnp.stack([w1, w2], axis=-1))


def kernel(hidden_states, weight):
    bsz, seq_len, h = hidden_states.shape
    x2d = hidden_states.reshape(-1, h).astype(jnp.float32)
    topk_idx, topk_weight = _route(x2d, weight.astype(jnp.float32))
    return (topk_idx, topk_weight)


# speedup vs baseline: 1.0779x; 1.0016x over previous
import functools

import jax
import jax.numpy as jnp
from jax import lax
from jax.experimental import pallas as pl
from jax.experimental.pallas import tpu as pltpu
from jax.experimental.pallas import tpu_sc as plsc

TOPK = 2
EPS = 1e-20
E = 8
LANES = 16


def _scores_kernel(x_ref, w_ref, st_ref):
    x = x_ref[...]
    w = w_ref[...]
    logits = jax.lax.dot_general(
        x, w, (((1,), (1,)), ((), ())),
        preferred_element_type=jnp.float32)
    m = jnp.max(logits, axis=-1, keepdims=True)
    e = jnp.exp(logits - m)
    s = e / jnp.sum(e, axis=-1, keepdims=True)
    tile = s.shape[0]
    wpt = tile // 256
    st_ref[...] = s.reshape(wpt, 256, E).transpose(0, 2, 1)


@functools.partial(jax.jit, static_argnames=("tile",))
def _scores_t(x2d, w, tile):
    S, H = x2d.shape
    wpt = tile // 256
    return pl.pallas_call(
        _scores_kernel,
        grid=(S // tile,),
        in_specs=[
            pl.BlockSpec((tile, H), lambda i: (i, 0)),
            pl.BlockSpec((E, H), lambda i: (0, 0)),
        ],
        out_specs=pl.BlockSpec((wpt, E, 256), lambda i: (i, 0, 0)),
        out_shape=jax.ShapeDtypeStruct((S // 256, E, 256), jnp.float32),
        compiler_params=pltpu.CompilerParams(
            dimension_semantics=("parallel",)),
    )(x2d, w)


def _make_sc_router(S):
    info = plsc.get_sparse_core_info()
    NC, NS = info.num_cores, info.num_subcores
    NW = NC * NS
    rows = S // NW
    chunks = rows // LANES

    mesh = plsc.VectorSubcoreMesh(core_axis_name="c", subcore_axis_name="s")

    @functools.partial(
        pl.kernel,
        mesh=mesh,
        out_type=[
            jax.ShapeDtypeStruct((S,), jnp.int32),
            jax.ShapeDtypeStruct((S,), jnp.int32),
            jax.ShapeDtypeStruct((S,), jnp.float32),
            jax.ShapeDtypeStruct((S,), jnp.float32),
        ],
        scratch_types=[
            pltpu.VMEM((E, rows), jnp.float32),
            pltpu.VMEM((rows,), jnp.int32),
            pltpu.VMEM((rows,), jnp.int32),
            pltpu.VMEM((rows,), jnp.float32),
            pltpu.VMEM((rows,), jnp.float32),
        ],
    )
    def _sc_router(st_hbm, i1_hbm, i2_hbm, w1_hbm, w2_hbm,
                   s_v, i1_v, i2_v, w1_v, w2_v):
        wid = lax.axis_index("s") * NC + lax.axis_index("c")
        base = wid * rows
        pltpu.sync_copy(st_hbm.at[wid], s_v)
        zeros = jnp.zeros((LANES,), jnp.int32)
        neg = jnp.full((LANES,), -jnp.inf, jnp.float32)
        evs = [jnp.full((LANES,), e, jnp.int32) for e in range(E)]
        for j in range(chunks):
            vs = [s_v[e, pl.ds(j * LANES, LANES)] for e in range(E)]
            m1 = vs[0]
            i1 = zeros
            for e in range(1, E):
                gt = vs[e] > m1
                m1 = jnp.where(gt, vs[e], m1)
                i1 = jnp.where(gt, evs[e], i1)
            m2 = neg
            i2 = zeros
            for e in range(E):
                cand = jnp.where(i1 == evs[e], neg, vs[e])
                gt = cand > m2
                m2 = jnp.where(gt, cand, m2)
                i2 = jnp.where(gt, evs[e], i2)
            inv = 1.0 / (m1 + m2 + EPS)
            sl = pl.ds(j * LANES, LANES)
            i1_v[sl] = i1
            i2_v[sl] = i2
            w1_v[sl] = m1 * inv
            w2_v[sl] = m2 * inv
        row_sl = pl.ds(base, rows)
        pltpu.sync_copy(i1_v, i1_hbm.at[row_sl])
        pltpu.sync_copy(i2_v, i2_hbm.at[row_sl])
        pltpu.sync_copy(w1_v, w1_hbm.at[row_sl])
        pltpu.sync_copy(w2_v, w2_hbm.at[row_sl])

    return _sc_router


@jax.jit
def _route(x2d, w):
    S = x2d.shape[0]
    st = _scores_t(x2d, w, tile=1024)
    i1, i2, w1, w2 = _make_sc_router(S)(st)
    return (jnp.stack([i1, i2], axis=-1), jnp.stack([w1, w2], axis=-1))


def kernel(hidden_states, weight):
    bsz, seq_len, h = hidden_states.shape
    x2d = hidden_states.reshape(-1, h).astype(jnp.float32)
    topk_idx, topk_weight = _route(x2d, weight.astype(jnp.float32))
    return (topk_idx, topk_weight)


# speedup vs baseline: 1.0797x; 1.0017x over previous
import functools

import jax
import jax.numpy as jnp
from jax import lax
from jax.experimental import pallas as pl
from jax.experimental.pallas import tpu as pltpu
from jax.experimental.pallas import tpu_sc as plsc

TOPK = 2
EPS = 1e-20
E = 8
LANES = 16


def _scores_kernel(x_ref, w_ref, st_ref):
    x = x_ref[...]
    w = w_ref[...]
    logits = jax.lax.dot_general(
        x, w, (((1,), (1,)), ((), ())),
        preferred_element_type=jnp.float32)
    m = jnp.max(logits, axis=-1, keepdims=True)
    e = jnp.exp(logits - m)
    s = e / jnp.sum(e, axis=-1, keepdims=True)
    tile = s.shape[0]
    wpt = tile // 256
    st_ref[...] = s.reshape(wpt, 256, E).transpose(0, 2, 1)


@functools.partial(jax.jit, static_argnames=("tile",))
def _scores_t(x2d, w, tile):
    S, H = x2d.shape
    wpt = tile // 256
    return pl.pallas_call(
        _scores_kernel,
        grid=(S // tile,),
        in_specs=[
            pl.BlockSpec((tile, H), lambda i: (i, 0)),
            pl.BlockSpec((E, H), lambda i: (0, 0)),
        ],
        out_specs=pl.BlockSpec((wpt, E, 256), lambda i: (i, 0, 0)),
        out_shape=jax.ShapeDtypeStruct((S // 256, E, 256), jnp.float32),
        compiler_params=pltpu.CompilerParams(
            dimension_semantics=("parallel",)),
    )(x2d, w)


def _make_sc_router(S):
    info = plsc.get_sparse_core_info()
    NC, NS = info.num_cores, info.num_subcores
    NW = NC * NS
    rows = S // NW
    chunks = rows // LANES

    mesh = plsc.VectorSubcoreMesh(core_axis_name="c", subcore_axis_name="s")

    @functools.partial(
        pl.kernel,
        mesh=mesh,
        out_type=[
            jax.ShapeDtypeStruct((S,), jnp.int32),
            jax.ShapeDtypeStruct((S,), jnp.int32),
            jax.ShapeDtypeStruct((S,), jnp.float32),
            jax.ShapeDtypeStruct((S,), jnp.float32),
        ],
        scratch_types=[
            pltpu.VMEM((E, rows), jnp.float32),
            pltpu.VMEM((rows,), jnp.int32),
            pltpu.VMEM((rows,), jnp.int32),
            pltpu.VMEM((rows,), jnp.float32),
            pltpu.VMEM((rows,), jnp.float32),
        ],
    )
    def _sc_router(st_hbm, i1_hbm, i2_hbm, w1_hbm, w2_hbm,
                   s_v, i1_v, i2_v, w1_v, w2_v):
        wid = lax.axis_index("s") * NC + lax.axis_index("c")
        base = wid * rows
        pltpu.sync_copy(st_hbm.at[wid], s_v)
        zeros = jnp.zeros((LANES,), jnp.int32)
        neg = jnp.full((LANES,), -jnp.inf, jnp.float32)
        evs = [jnp.full((LANES,), e, jnp.int32) for e in range(E)]
        for j in range(chunks):
            vs = [s_v[e, pl.ds(j * LANES, LANES)] for e in range(E)]
            m1 = vs[0]
            i1 = zeros
            for e in range(1, E):
                gt = vs[e] > m1
                m1 = jnp.where(gt, vs[e], m1)
                i1 = jnp.where(gt, evs[e], i1)
            m2 = neg
            i2 = zeros
            for e in range(E):
                cand = jnp.where(i1 == evs[e], neg, vs[e])
                gt = cand > m2
                m2 = jnp.where(gt, cand, m2)
                i2 = jnp.where(gt, evs[e], i2)
            inv = 1.0 / (m1 + m2 + EPS)
            sl = pl.ds(j * LANES, LANES)
            i1_v[sl] = i1
            i2_v[sl] = i2
            w1_v[sl] = m1 * inv
            w2_v[sl] = m2 * inv
        row_sl = pl.ds(base, rows)
        pltpu.sync_copy(i1_v, i1_hbm.at[row_sl])
        pltpu.sync_copy(i2_v, i2_hbm.at[row_sl])
        pltpu.sync_copy(w1_v, w1_hbm.at[row_sl])
        pltpu.sync_copy(w2_v, w2_hbm.at[row_sl])

    return _sc_router


@jax.jit
def _route(x2d, w):
    S = x2d.shape[0]
    st = _scores_t(x2d, w, tile=512)
    i1, i2, w1, w2 = _make_sc_router(S)(st)
    return (jnp.stack([i1, i2], axis=-1), jnp.stack([w1, w2], axis=-1))


def kernel(hidden_states, weight):
    bsz, seq_len, h = hidden_states.shape
    x2d = hidden_states.reshape(-1, h).astype(jnp.float32)
    topk_idx, topk_weight = _route(x2d, weight.astype(jnp.float32))
    return (topk_idx, topk_weight)
